# baseline (speedup 1.0000x reference)
import functools

import jax
import jax.numpy as jnp
from jax import lax
from jax.experimental import pallas as pl
from jax.experimental.pallas import tpu as pltpu
from jax.experimental.pallas import tpu_sc as plsc

_MAX_LEN = 200
_OBS = 128
_NC = 2
_NS = 16
_LANES = 16
_MASK_PAD = 256


def _fill_body(x_ref, buf_ref):
    row = lax.broadcasted_iota(jnp.int32, buf_ref.shape, 1)
    buf_ref[...] = jnp.where(row == 0, x_ref[...][:, None, :], 0.0)


def _tc_fill(x):
    batch = x.shape[0]
    bblk = 64
    return pl.pallas_call(
        _fill_body,
        grid=(batch // bblk,),
        in_specs=[pl.BlockSpec((bblk, _OBS), lambda i: (i, 0))],
        out_specs=pl.BlockSpec((bblk, _MAX_LEN, _OBS), lambda i: (i, 0, 0)),
        out_shape=jax.ShapeDtypeStruct((batch, _MAX_LEN, _OBS), x.dtype),
        compiler_params=pltpu.CompilerParams(
            dimension_semantics=("parallel",),
        ),
    )(x)


def _make_sc_mask():
    mesh = plsc.VectorSubcoreMesh(
        core_axis_name="c", subcore_axis_name="s",
        num_cores=_NC, num_subcores=_NS,
    )

    @functools.partial(
        pl.kernel,
        mesh=mesh,
        out_type=jax.ShapeDtypeStruct((_MASK_PAD,), jnp.int32),
        scratch_types=[pltpu.VMEM((_MASK_PAD,), jnp.int32)],
    )
    def sc_mask(out_hbm, mask_v):
        wid = lax.axis_index("s") * _NC + lax.axis_index("c")

        @pl.when(wid == 0)
        def _():
            lane = lax.iota(jnp.int32, _LANES)
            for j in range(_MASK_PAD // _LANES):
                pos = j * _LANES + lane
                mask_v[pl.ds(j * _LANES, _LANES)] = jnp.where(
                    pos >= _MAX_LEN - 1, 1, 0)
            pltpu.sync_copy(mask_v, out_hbm)

    return sc_mask


def kernel(x):
    buf = _tc_fill(x)
    mask_padded = _make_sc_mask()()
    return buf, (mask_padded[:_MAX_LEN] != 0)


# speedup vs baseline: 1.0043x; 1.0043x over previous
import functools

import jax
import jax.numpy as jnp
from jax import lax
from jax.experimental import pallas as pl
from jax.experimental.pallas import tpu as pltpu
from jax.experimental.pallas import tpu_sc as plsc

_MAX_LEN = 200
_OBS = 128
_NC = 2
_NS = 16
_LANES = 16
_MASK_PAD = 256


def _fill_body(x_ref, buf_ref):
    row = lax.broadcasted_iota(jnp.int32, buf_ref.shape, 1)
    buf_ref[...] = jnp.where(row == 0, x_ref[...][:, None, :], 0.0)


def _tc_fill(x):
    batch = x.shape[0]
    bblk = 64
    return pl.pallas_call(
        _fill_body,
        grid=(batch // bblk,),
        in_specs=[pl.BlockSpec((bblk, _OBS), lambda i: (i, 0))],
        out_specs=pl.BlockSpec((bblk, _MAX_LEN, _OBS), lambda i: (i, 0, 0)),
        out_shape=jax.ShapeDtypeStruct((batch, _MAX_LEN, _OBS), x.dtype),
        compiler_params=pltpu.CompilerParams(
            dimension_semantics=("parallel",),
        ),
    )(x)


def _make_sc_mask():
    mesh = plsc.VectorSubcoreMesh(
        core_axis_name="c", subcore_axis_name="s",
        num_cores=_NC, num_subcores=_NS,
    )

    @functools.partial(
        pl.kernel,
        mesh=mesh,
        out_type=jax.ShapeDtypeStruct((_MASK_PAD,), jnp.int32),
        scratch_types=[pltpu.VMEM((_MASK_PAD,), jnp.int32)],
    )
    def sc_mask(out_hbm, mask_v):
        wid = lax.axis_index("s") * _NC + lax.axis_index("c")

        @pl.when(wid == 0)
        def _():
            lane = lax.iota(jnp.int32, _LANES)
            for j in range(_MASK_PAD // _LANES):
                pos = j * _LANES + lane
                mask_v[pl.ds(j * _LANES, _LANES)] = jnp.where(
                    pos >= _MAX_LEN - 1, 1, 0)
            pltpu.sync_copy(mask_v, out_hbm)

    return sc_mask


def kernel(x):
    mask_padded = _make_sc_mask()()
    buf = _tc_fill(x)
    return buf, (mask_padded[:_MAX_LEN] != 0)
